# baseline (speedup 1.0000x reference)
import functools

import jax
import jax.numpy as jnp
from jax import lax
from jax.experimental import pallas as pl
from jax.experimental.pallas import tpu as pltpu
from jax.experimental.pallas import tpu_sc as plsc

_NC = 2
_NS = 16
_NW = _NC * _NS
_CH = 16
_L = 16

_BS = 512
_TCB = 3


def _sc_lookup_add(x, idx, emb, batch):
    B, S, D = x.shape
    rows_per_w = S // _NW
    n_ch = rows_per_w // _CH
    mesh = plsc.VectorSubcoreMesh(
        core_axis_name="c", subcore_axis_name="s",
        num_cores=_NC, num_subcores=_NS)

    @functools.partial(
        pl.kernel,
        out_type=jax.ShapeDtypeStruct((S, D), jnp.float32),
        mesh=mesh,
        scratch_types=[
            pltpu.VMEM((rows_per_w,), jnp.int32),
            pltpu.VMEM((_CH, D), jnp.float32),
            pltpu.VMEM((_CH, D), jnp.float32),
            pltpu.VMEM((_CH, D), jnp.float32),
            pltpu.VMEM((_CH, D), jnp.float32),
            pltpu.SemaphoreType.DMA,
            pltpu.SemaphoreType.DMA,
            pltpu.SemaphoreType.DMA,
            pltpu.SemaphoreType.DMA,
            pltpu.SemaphoreType.DMA,
            pltpu.SemaphoreType.DMA,
        ],
    )
    def body(x_hbm, idx_hbm, emb_hbm, out_hbm,
             idx_v, xb0, xb1, eb0, eb1, sx0, sx1, se0, se1, sw0, sw1):
        xb = (xb0, xb1)
        eb = (eb0, eb1)
        sx = (sx0, sx1)
        se = (se0, se1)
        sw = (sw0, sw1)
        wid = lax.axis_index("s") * _NC + lax.axis_index("c")
        base = wid * rows_per_w
        pltpu.sync_copy(idx_hbm.at[pl.ds(base, rows_per_w)], idx_v)

        def start_gather(ch, k):
            r0 = base + ch * _CH
            pltpu.async_copy(x_hbm.at[batch, pl.ds(r0, _CH)], xb[k], sx[k])
            pltpu.async_copy(
                emb_hbm.at[idx_v.at[pl.ds(ch * _CH, _CH)]], eb[k], se[k])

        def add_chunk(k):
            def row(r, carry):
                for d in range(D // _L):
                    sl = pl.ds(d * _L, _L)
                    xb[k][r, sl] = xb[k][r, sl] + eb[k][r, sl]
                return carry
            lax.fori_loop(0, _CH, row, 0)

        start_gather(0, 0)
        def pair(p, carry):
            for k in (0, 1):
                ch = p * 2 + k
                pltpu.make_async_copy(x_hbm.at[batch, pl.ds(base, _CH)],
                                      xb[k], sx[k]).wait()
                pltpu.make_async_copy(emb_hbm.at[idx_v.at[pl.ds(0, _CH)]],
                                      eb[k], se[k]).wait()
                @pl.when(ch >= 1)
                def _():
                    pltpu.make_async_copy(
                        xb[1 - k], out_hbm.at[pl.ds(base, _CH)],
                        sw[1 - k]).wait()
                @pl.when(ch + 1 < n_ch)
                def _():
                    start_gather(ch + 1, 1 - k)
                add_chunk(k)
                pltpu.async_copy(xb[k],
                                 out_hbm.at[pl.ds(base + ch * _CH, _CH)],
                                 sw[k])
            return carry
        lax.fori_loop(0, n_ch // 2, pair, 0)
        pltpu.make_async_copy(xb[1], out_hbm.at[pl.ds(base, _CH)], sw[1]).wait()

    return body(x, idx, emb)


def _add_body(x_ref, emb_ref, out_ref):
    out_ref[...] = x_ref[...] + emb_ref[...][None, :, :]


def _tc_add(x, emb):
    B, S, D = x.shape
    return pl.pallas_call(
        _add_body,
        grid=(S // _BS, _TCB),
        in_specs=[
            pl.BlockSpec((1, _BS, D), lambda i, b: (b, i, 0)),
            pl.BlockSpec((_BS, D), lambda i, b: (i, 0)),
        ],
        out_specs=pl.BlockSpec((1, _BS, D), lambda i, b: (b, i, 0)),
        out_shape=jax.ShapeDtypeStruct((_TCB, S, D), x.dtype),
    )(x, emb)


def kernel(x, pos_ids, emb):
    B, S, D = x.shape
    idx = pos_ids[0, :S].astype(jnp.int32)
    out_tc = _tc_add(x, emb)
    out_sc = _sc_lookup_add(x, idx, emb, B - 1)
    return jnp.concatenate([out_tc, out_sc[None]], axis=0)


# speedup vs baseline: 1.5253x; 1.5253x over previous
import functools

import jax
import jax.numpy as jnp
from jax import lax
from jax.experimental import pallas as pl
from jax.experimental.pallas import tpu as pltpu
from jax.experimental.pallas import tpu_sc as plsc

_NC = 2
_NS = 16
_NW = _NC * _NS
_CH = 32

_BS = 512


def _sc_gather(idx, emb):
    S = idx.shape[0]
    D = emb.shape[1]
    rows_per_w = S // _NW
    n_ch = rows_per_w // _CH
    mesh = plsc.VectorSubcoreMesh(
        core_axis_name="c", subcore_axis_name="s",
        num_cores=_NC, num_subcores=_NS)

    @functools.partial(
        pl.kernel,
        out_type=jax.ShapeDtypeStruct((S, D), jnp.float32),
        mesh=mesh,
        scratch_types=[
            pltpu.VMEM((rows_per_w,), jnp.int32),
            pltpu.VMEM((_CH, D), jnp.float32),
            pltpu.VMEM((_CH, D), jnp.float32),
            pltpu.SemaphoreType.DMA,
            pltpu.SemaphoreType.DMA,
            pltpu.SemaphoreType.DMA,
            pltpu.SemaphoreType.DMA,
        ],
    )
    def body(idx_hbm, emb_hbm, pe_hbm,
             idx_v, rb0, rb1, sg0, sg1, sw0, sw1):
        rb = (rb0, rb1)
        sg = (sg0, sg1)
        sw = (sw0, sw1)
        wid = lax.axis_index("s") * _NC + lax.axis_index("c")
        base = wid * rows_per_w
        pltpu.sync_copy(idx_hbm.at[pl.ds(base, rows_per_w)], idx_v)

        def start_gather(ch, k):
            pltpu.async_copy(
                emb_hbm.at[idx_v.at[pl.ds(ch * _CH, _CH)]], rb[k], sg[k])

        start_gather(0, 0)
        def pair(p, carry):
            for k in (0, 1):
                ch = p * 2 + k
                pltpu.make_async_copy(emb_hbm.at[idx_v.at[pl.ds(0, _CH)]],
                                      rb[k], sg[k]).wait()
                @pl.when(ch >= 1)
                def _():
                    pltpu.make_async_copy(
                        rb[1 - k], pe_hbm.at[pl.ds(base, _CH)],
                        sw[1 - k]).wait()
                @pl.when(ch + 1 < n_ch)
                def _():
                    start_gather(ch + 1, 1 - k)
                pltpu.async_copy(rb[k],
                                 pe_hbm.at[pl.ds(base + ch * _CH, _CH)],
                                 sw[k])
            return carry
        lax.fori_loop(0, n_ch // 2, pair, 0)
        pltpu.make_async_copy(rb[1], pe_hbm.at[pl.ds(base, _CH)], sw[1]).wait()

    return body(idx, emb)


def _add_body(x_ref, pe_ref, out_ref):
    out_ref[...] = x_ref[...] + pe_ref[...][None, :, :]


def _tc_add(x, pe):
    B, S, D = x.shape
    return pl.pallas_call(
        _add_body,
        grid=(S // _BS,),
        in_specs=[
            pl.BlockSpec((B, _BS, D), lambda i: (0, i, 0)),
            pl.BlockSpec((_BS, D), lambda i: (i, 0)),
        ],
        out_specs=pl.BlockSpec((B, _BS, D), lambda i: (0, i, 0)),
        out_shape=jax.ShapeDtypeStruct((B, S, D), x.dtype),
    )(x, pe)


def kernel(x, pos_ids, emb):
    B, S, D = x.shape
    idx = pos_ids[0, :S].astype(jnp.int32)
    pe = _sc_gather(idx, emb)
    return _tc_add(x, pe)
